# Initial kernel scaffold; baseline (speedup 1.0000x reference)
import jax
import jax.numpy as jnp
from jax.experimental import pallas as pl


def kernel(x, edge_index, edge_attr, params):
    raise NotImplementedError("write your pallas kernel here")


# speedup vs baseline: 11.6793x; 11.6793x over previous
import functools

import jax
import jax.numpy as jnp
from jax import lax
from jax.experimental import pallas as pl
from jax.experimental.pallas import tpu as pltpu

NN = 750
NP = 768
E_CH = 512
N_CHUNKS = 47

_F32 = jnp.float32
_BF16 = jnp.bfloat16


def _pcall(*args, **kwargs):
    return pl.pallas_call(*args, **kwargs)


def _bdot(a, b):
    return jnp.dot(a.astype(_BF16), b.astype(_BF16),
                   preferred_element_type=_F32)


def _build_a_body(row_ref, col_ref, attr_ref, out_ref, s_acc):
    i = pl.program_id(0)

    @pl.when(i == 0)
    def _():
        s_acc[...] = jnp.zeros_like(s_acc)

    r = row_ref[0]
    c = col_ref[0]
    a = attr_ref[0]
    iota_n = lax.broadcasted_iota(jnp.int32, (E_CH, NP), 1)
    ohr = jnp.where(iota_n == r, 1.0, 0.0).astype(_F32)
    ohc = jnp.where(iota_n == c, 1.0, 0.0).astype(_F32)
    s_acc[...] += lax.dot_general(
        ohr * a, ohc, (((0,), (0,)), ((), ())), preferred_element_type=_F32, precision=lax.Precision.HIGHEST)

    @pl.when(i == N_CHUNKS - 1)
    def _():
        s = s_acc[...]
        deg = jnp.sum(s, axis=1, keepdims=True)
        dis = jnp.where(deg > 0, lax.rsqrt(jnp.where(deg > 0, deg, 1.0)), 0.0)
        row_scaled = -(dis * s)
        eq = (lax.broadcasted_iota(jnp.int32, (NP, NP), 0)
              == lax.broadcasted_iota(jnp.int32, (NP, NP), 1))
        diag = jnp.where(eq, dis, 0.0)
        a_full = jnp.dot(row_scaled, diag, preferred_element_type=_F32, precision=lax.Precision.HIGHEST)
        out_ref[...] = a_full[:NN, :NN]


def _build_a(row3, col3, attr3):
    espec = pl.BlockSpec((1, E_CH, 1), lambda i: (i, 0, 0))
    return _pcall(
        _build_a_body,
        grid=(N_CHUNKS,),
        in_specs=[espec, espec, espec],
        out_specs=pl.BlockSpec((NN, NN), lambda i: (0, 0)),
        out_shape=jax.ShapeDtypeStruct((NN, NN), _F32),
        scratch_shapes=[pltpu.VMEM((NP, NP), _F32)],
    )(row3, col3, attr3)


def _m2_body(a_ref, out_ref):
    a = a_ref[...]
    eye = jnp.where(
        lax.broadcasted_iota(jnp.int32, (NN, NN), 0)
        == lax.broadcasted_iota(jnp.int32, (NN, NN), 1), 1.0, 0.0)
    out_ref[...] = 2.0 * jnp.dot(a, a, preferred_element_type=_F32, precision=lax.Precision.HIGHEST) - eye


def _m2_of(a):
    return _pcall(_m2_body, out_shape=jax.ShapeDtypeStruct((NN, NN), _F32))(a)


def _make_tconv(B, T_in, Cin, Cout, k, has_norm, has_stats):
    T_out = T_in - k + 1
    BN = B * NN

    def body(*refs):
        x_ref, wp_ref, wq_ref, wr_ref, bp_ref, bq_ref, br_ref = refs[:7]
        pos = 7
        if has_norm:
            scale_ref, shift_ref = refs[pos:pos + 2]
            pos += 2
        out_ref = refs[pos]
        pos += 1
        if has_stats:
            stats_ref = refs[pos]
            pos += 1
        pacc, qacc, racc = refs[pos:pos + 3]

        t = pl.program_id(0)
        j = pl.program_id(1)

        if has_stats:
            @pl.when(jnp.logical_and(t == 0, j == 0))
            def _():
                stats_ref[...] = jnp.zeros_like(stats_ref)

        @pl.when(j == 0)
        def _():
            pacc[...] = jnp.zeros_like(pacc)
            qacc[...] = jnp.zeros_like(qacc)
            racc[...] = jnp.zeros_like(racc)

        x = x_ref[...]
        if has_norm:
            x = x * scale_ref[...][None, None] + shift_ref[...][None, None]
        x2 = x.reshape(BN, Cin)
        pacc[...] += _bdot(x2, wp_ref[0])
        qacc[...] += _bdot(x2, wq_ref[0])
        racc[...] += _bdot(x2, wr_ref[0])

        @pl.when(j == k - 1)
        def _():
            p = pacc[...] + bp_ref[...]
            q = jax.nn.sigmoid(qacc[...] + bq_ref[...])
            r = racc[...] + br_ref[...]
            y = jnp.maximum(p * q + r, 0.0)
            out_ref[...] = y.reshape(B, 1, NN, Cout)
            if has_stats:
                y3 = y.reshape(B, NN, Cout)
                stats_ref[0] += jnp.sum(y3, axis=0)
                stats_ref[1] += jnp.sum(y3 * y3, axis=0)

    in_specs = [
        pl.BlockSpec((B, 1, NN, Cin), lambda t, j: (0, t + j, 0, 0)),
        pl.BlockSpec((1, Cin, Cout), lambda t, j: (j, 0, 0)),
        pl.BlockSpec((1, Cin, Cout), lambda t, j: (j, 0, 0)),
        pl.BlockSpec((1, Cin, Cout), lambda t, j: (j, 0, 0)),
        pl.BlockSpec((1, Cout), lambda t, j: (0, 0)),
        pl.BlockSpec((1, Cout), lambda t, j: (0, 0)),
        pl.BlockSpec((1, Cout), lambda t, j: (0, 0)),
    ]
    if has_norm:
        in_specs += [pl.BlockSpec((NN, 1), lambda t, j: (0, 0)),
                     pl.BlockSpec((NN, 1), lambda t, j: (0, 0))]
    out_specs = pl.BlockSpec((B, 1, NN, Cout), lambda t, j: (0, t, 0, 0))
    out_shape = jax.ShapeDtypeStruct((B, T_out, NN, Cout), _F32)
    if has_stats:
        out_specs = [out_specs, pl.BlockSpec((2, NN, Cout), lambda t, j: (0, 0, 0))]
        out_shape = [out_shape, jax.ShapeDtypeStruct((2, NN, Cout), _F32)]

    return functools.partial(
        _pcall,
        body,
        grid=(T_out, k),
        in_specs=in_specs,
        out_specs=out_specs,
        out_shape=out_shape,
        scratch_shapes=[pltpu.VMEM((BN, Cout), _F32)] * 3,
    )


def _tconv(x, p, norm, stats):
    B, T_in, _, Cin = x.shape
    Cout, _, k = p["Wp"].shape
    call = _make_tconv(B, T_in, Cin, Cout, k, norm is not None, stats)()
    args = [x,
            jnp.transpose(p["Wp"], (2, 1, 0)),
            jnp.transpose(p["Wq"], (2, 1, 0)),
            jnp.transpose(p["Wr"], (2, 1, 0)),
            p["bp"].reshape(1, Cout),
            p["bq"].reshape(1, Cout),
            p["br"].reshape(1, Cout)]
    if norm is not None:
        args += [norm[0], norm[1]]
    return call(*args)


def _make_cheb(BT, C):
    def body(x_ref, a_ref, m2_ref, w_ref, b_ref, out_ref):
        x = x_ref[0]
        t1 = jnp.dot(a_ref[...], x, preferred_element_type=_F32,
                     precision=lax.Precision.HIGHEST)
        t2 = jnp.dot(m2_ref[...], x, preferred_element_type=_F32,
                     precision=lax.Precision.HIGHEST)
        y = (_bdot(x, w_ref[0]) + _bdot(t1, w_ref[1]) + _bdot(t2, w_ref[2])
             + b_ref[...])
        out_ref[0] = jnp.maximum(y, 0.0)

    return functools.partial(
        _pcall,
        body,
        grid=(BT,),
        in_specs=[
            pl.BlockSpec((1, NN, C), lambda i: (i, 0, 0)),
            pl.BlockSpec((NN, NN), lambda i: (0, 0)),
            pl.BlockSpec((NN, NN), lambda i: (0, 0)),
            pl.BlockSpec((3, C, C), lambda i: (0, 0, 0)),
            pl.BlockSpec((1, C), lambda i: (0, 0)),
        ],
        out_specs=pl.BlockSpec((1, NN, C), lambda i: (i, 0, 0)),
        out_shape=jax.ShapeDtypeStruct((BT, NN, C), _F32),
    )


def _cheb(x, a, m2, w, b):
    B, T, _, C = x.shape
    call = _make_cheb(B * T, C)()
    y = call(x.reshape(B * T, NN, C), a, m2, w, b.reshape(1, C))
    return y.reshape(B, T, NN, C)


def _make_bnfinal(cnt):
    inv_cnt = 1.0 / float(cnt)

    def body(stats_ref, g_ref, b_ref, scale_ref, shift_ref):
        mean = jnp.sum(stats_ref[0], axis=1, keepdims=True) * inv_cnt
        var = jnp.sum(stats_ref[1], axis=1, keepdims=True) * inv_cnt - mean * mean
        inv = lax.rsqrt(var + 1e-5)
        scale = g_ref[...] * inv
        scale_ref[...] = scale
        shift_ref[...] = b_ref[...] - mean * scale

    return functools.partial(
        _pcall,
        body,
        out_shape=[jax.ShapeDtypeStruct((NN, 1), _F32),
                   jax.ShapeDtypeStruct((NN, 1), _F32)],
    )


def _bnfinal(stats, g, b, cnt):
    return _make_bnfinal(cnt)()(stats, g.reshape(NN, 1), b.reshape(NN, 1))


def _make_head(B, T, C):
    def body(x_ref, scale_ref, shift_ref, w1t_ref, b1_ref, w2t_ref, b2_ref,
             out_ref):
        x = x_ref[0]
        xn = x * scale_ref[...][None] + shift_ref[...][None]
        y1 = _bdot(xn.reshape(T * NN, C), w1t_ref[...]) + b1_ref[...]
        y2 = _bdot(y1, w2t_ref[...]) + b2_ref[...]
        out_ref[0] = y2.reshape(T, NN, 7)

    return functools.partial(
        _pcall,
        body,
        grid=(B,),
        in_specs=[
            pl.BlockSpec((1, T, NN, C), lambda b: (b, 0, 0, 0)),
            pl.BlockSpec((NN, 1), lambda b: (0, 0)),
            pl.BlockSpec((NN, 1), lambda b: (0, 0)),
            pl.BlockSpec((C, 32), lambda b: (0, 0)),
            pl.BlockSpec((1, 32), lambda b: (0, 0)),
            pl.BlockSpec((32, 7), lambda b: (0, 0)),
            pl.BlockSpec((1, 7), lambda b: (0, 0)),
        ],
        out_specs=pl.BlockSpec((1, T, NN, 7), lambda b: (b, 0, 0, 0)),
        out_shape=jax.ShapeDtypeStruct((B, T, NN, 7), _F32),
    )


def kernel(x, edge_index, edge_attr, params):
    row, col = edge_index[0], edge_index[1]
    pad = N_CHUNKS * E_CH - row.shape[0]
    row3 = jnp.pad(row, (0, pad)).reshape(N_CHUNKS, E_CH, 1)
    col3 = jnp.pad(col, (0, pad)).reshape(N_CHUNKS, E_CH, 1)
    attr3 = jnp.pad(edge_attr, (0, pad)).reshape(N_CHUNKS, E_CH, 1)

    a = _build_a(row3, col3, attr3)
    m2 = _m2_of(a)

    t = x
    norm = None
    for bname in ("b1", "b2", "b3"):
        p = params[bname]
        t = _tconv(t, p["tc1"], norm=norm, stats=False)
        t = _cheb(t, a, m2, p["cheb_W"], p["cheb_b"])
        t, stats = _tconv(t, p["tc2"], norm=None, stats=True)
        cnt = t.shape[0] * t.shape[1] * t.shape[3]
        norm = _bnfinal(stats, p["bn_g"], p["bn_b"], cnt)

    B, T, _, C = t.shape
    head = _make_head(B, T, C)()
    return head(t, norm[0], norm[1],
                jnp.transpose(params["fc1_W"]),
                params["fc1_b"].reshape(1, 32),
                jnp.transpose(params["fc2_W"]),
                params["fc2_b"].reshape(1, 7))


# speedup vs baseline: 12.5175x; 1.0718x over previous
import functools

import jax
import jax.numpy as jnp
from jax import lax
from jax.experimental import pallas as pl
from jax.experimental.pallas import tpu as pltpu
from jax.experimental.pallas import tpu_sc as plsc

NN = 750
NP = 768
E_CH = 512
N_CHUNKS = 47

_F32 = jnp.float32
_BF16 = jnp.bfloat16


def _pcall(*args, **kwargs):
    return pl.pallas_call(*args, **kwargs)


def _bdot(a, b):
    return jnp.dot(a.astype(_BF16), b.astype(_BF16),
                   preferred_element_type=_F32)


def _build_a_body(row_ref, col_ref, attr_ref, out_ref, s_acc):
    i = pl.program_id(0)

    @pl.when(i == 0)
    def _():
        s_acc[...] = jnp.zeros_like(s_acc)

    r = row_ref[0]
    c = col_ref[0]
    a = attr_ref[0]
    iota_n = lax.broadcasted_iota(jnp.int32, (E_CH, NP), 1)
    ohr = jnp.where(iota_n == r, 1.0, 0.0).astype(_F32)
    ohc = jnp.where(iota_n == c, 1.0, 0.0).astype(_F32)
    s_acc[...] += lax.dot_general(
        ohr * a, ohc, (((0,), (0,)), ((), ())), preferred_element_type=_F32, precision=lax.Precision.HIGHEST)

    @pl.when(i == N_CHUNKS - 1)
    def _():
        s = s_acc[...]
        deg = jnp.sum(s, axis=1, keepdims=True)
        dis = jnp.where(deg > 0, lax.rsqrt(jnp.where(deg > 0, deg, 1.0)), 0.0)
        row_scaled = -(dis * s)
        eq = (lax.broadcasted_iota(jnp.int32, (NP, NP), 0)
              == lax.broadcasted_iota(jnp.int32, (NP, NP), 1))
        diag = jnp.where(eq, dis, 0.0)
        a_full = jnp.dot(row_scaled, diag, preferred_element_type=_F32, precision=lax.Precision.HIGHEST)
        out_ref[...] = a_full[:NN, :NN]


def _build_a(row3, col3, attr3):
    espec = pl.BlockSpec((1, E_CH, 1), lambda i: (i, 0, 0))
    return _pcall(
        _build_a_body,
        grid=(N_CHUNKS,),
        in_specs=[espec, espec, espec],
        out_specs=pl.BlockSpec((NN, NN), lambda i: (0, 0)),
        out_shape=jax.ShapeDtypeStruct((NN, NN), _F32),
        scratch_shapes=[pltpu.VMEM((NP, NP), _F32)],
    )(row3, col3, attr3)


_EPT = 1504
_NVR = _EPT // 16
_ROWS_PT = 48
_SROW = 752
_SROWS = 768


def _sc_build_body(row_hbm, col_hbm, attr_hbm, s_out, deg_out,
                   rowv, colv, attrv, idx2d, idxd, val2d, zbuf, sbuf,
                   s_shf, deg_sh):
    cid = lax.axis_index("c")
    sid = lax.axis_index("s")
    r0 = sid * _ROWS_PT

    @pl.when(cid == 0)
    def _phase_zero():
        for q in range(48):
            zbuf[pl.ds(q * 16, 16)] = jnp.zeros((16,), _F32)
        pltpu.sync_copy(zbuf.at[pl.ds(0, _ROWS_PT)],
                        deg_sh.at[pl.ds(r0, _ROWS_PT)])

        def zrow(i, _):
            r = r0 + i
            pltpu.sync_copy(zbuf.at[pl.ds(0, _SROW)],
                            s_shf.at[pl.ds(r * _SROW, _SROW)])
            return 0
        lax.fori_loop(0, _ROWS_PT, zrow, 0)

    plsc.subcore_barrier()

    @pl.when(cid == 0)
    def _phase_scatter():
        base = sid * _EPT
        pltpu.sync_copy(row_hbm.at[pl.ds(base, _EPT)], rowv)
        pltpu.sync_copy(col_hbm.at[pl.ds(base, _EPT)], colv)
        pltpu.sync_copy(attr_hbm.at[pl.ds(base, _EPT)], attrv)
        for q in range(2):
            idx2d[11, pl.ds(96 + q * 16, 16)] = jnp.zeros((16,), jnp.int32)
            idxd[11, pl.ds(96 + q * 16, 16)] = jnp.zeros((16,), jnp.int32)
            val2d[11, pl.ds(96 + q * 16, 16)] = jnp.zeros((16,), _F32)
        for v in range(_NVR):
            r16 = rowv[pl.ds(v * 16, 16)]
            c16 = colv[pl.ds(v * 16, 16)]
            a16 = attrv[pl.ds(v * 16, 16)]
            jr, jc = v // 8, (v % 8) * 16
            idx2d[jr, pl.ds(jc, 16)] = r16 * _SROW + c16
            idxd[jr, pl.ds(jc, 16)] = r16
            val2d[jr, pl.ds(jc, 16)] = a16
        for jrow in range(12):
            pltpu.sync_copy(val2d.at[jrow], s_shf.at[idx2d.at[jrow]], add=True)
        for jrow in range(12):
            pltpu.sync_copy(val2d.at[jrow], deg_sh.at[idxd.at[jrow]], add=True)

    plsc.subcore_barrier()

    @pl.when(cid == 0)
    def _phase_out():
        pltpu.sync_copy(deg_sh.at[pl.ds(r0, _ROWS_PT)],
                        zbuf.at[pl.ds(0, _ROWS_PT)])
        pltpu.sync_copy(zbuf.at[pl.ds(0, _ROWS_PT)],
                        deg_out.at[pl.ds(r0, _ROWS_PT)])
        blk = 8 * _SROW
        for k in range(_ROWS_PT // 8):
            off = r0 * _SROW + k * blk
            pltpu.sync_copy(s_shf.at[pl.ds(off, blk)], sbuf)
            pltpu.sync_copy(sbuf, s_out.at[pl.ds(off, blk)])


def _build_a_sc(row_p, col_p, attr_p):
    mesh = plsc.VectorSubcoreMesh(core_axis_name="c", subcore_axis_name="s")
    f = functools.partial(
        pl.kernel,
        mesh=mesh,
        out_type=[jax.ShapeDtypeStruct((_SROWS * _SROW,), _F32),
                  jax.ShapeDtypeStruct((_SROWS,), _F32)],
        scratch_types=[
            pltpu.VMEM((_EPT,), jnp.int32),
            pltpu.VMEM((_EPT,), jnp.int32),
            pltpu.VMEM((_EPT,), _F32),
            pltpu.VMEM((12, 128), jnp.int32),
            pltpu.VMEM((12, 128), jnp.int32),
            pltpu.VMEM((12, 128), _F32),
            pltpu.VMEM((768,), _F32),
            pltpu.VMEM((8 * _SROW,), _F32),
            pltpu.VMEM_SHARED((_SROWS * _SROW,), _F32),
            pltpu.VMEM_SHARED((_SROWS,), _F32),
        ],
    )(_sc_build_body)
    return f(row_p, col_p, attr_p)


def _finish_body(s_ref, degc_ref, degr_ref, a_ref, m2_ref):
    s = s_ref[...][:NN, :NN]
    dc = degc_ref[...]
    dr = degr_ref[...][:, :NN]
    disc = jnp.where(dc > 0, lax.rsqrt(jnp.where(dc > 0, dc, 1.0)), 0.0)
    disr = jnp.where(dr > 0, lax.rsqrt(jnp.where(dr > 0, dr, 1.0)), 0.0)
    a = -(disc * s * disr)
    a_ref[...] = a
    eye = jnp.where(
        lax.broadcasted_iota(jnp.int32, (NN, NN), 0)
        == lax.broadcasted_iota(jnp.int32, (NN, NN), 1), 1.0, 0.0)
    m2_ref[...] = 2.0 * jnp.dot(a, a, preferred_element_type=_F32, precision=lax.Precision.HIGHEST) - eye


def _finish_a(s2d, degc, degr):
    return _pcall(
        _finish_body,
        out_shape=[jax.ShapeDtypeStruct((NN, NN), _F32),
                   jax.ShapeDtypeStruct((NN, NN), _F32)],
    )(s2d, degc, degr)


def _make_tconv(B, T_in, Cin, Cout, k, has_norm, has_stats):
    T_out = T_in - k + 1
    BN = B * NN

    def body(*refs):
        x_ref, wp_ref, wq_ref, wr_ref, bp_ref, bq_ref, br_ref = refs[:7]
        pos = 7
        if has_norm:
            scale_ref, shift_ref = refs[pos:pos + 2]
            pos += 2
        out_ref = refs[pos]
        pos += 1
        if has_stats:
            stats_ref = refs[pos]
            pos += 1
        pacc, qacc, racc = refs[pos:pos + 3]

        t = pl.program_id(0)
        j = pl.program_id(1)

        if has_stats:
            @pl.when(jnp.logical_and(t == 0, j == 0))
            def _():
                stats_ref[...] = jnp.zeros_like(stats_ref)

        @pl.when(j == 0)
        def _():
            pacc[...] = jnp.zeros_like(pacc)
            qacc[...] = jnp.zeros_like(qacc)
            racc[...] = jnp.zeros_like(racc)

        x = x_ref[...]
        if has_norm:
            x = x * scale_ref[...][None, None] + shift_ref[...][None, None]
        x2 = x.reshape(BN, Cin)
        pacc[...] += _bdot(x2, wp_ref[0])
        qacc[...] += _bdot(x2, wq_ref[0])
        racc[...] += _bdot(x2, wr_ref[0])

        @pl.when(j == k - 1)
        def _():
            p = pacc[...] + bp_ref[...]
            q = jax.nn.sigmoid(qacc[...] + bq_ref[...])
            r = racc[...] + br_ref[...]
            y = jnp.maximum(p * q + r, 0.0)
            out_ref[...] = y.reshape(B, 1, NN, Cout)
            if has_stats:
                y3 = y.reshape(B, NN, Cout)
                stats_ref[0] += jnp.sum(y3, axis=0)
                stats_ref[1] += jnp.sum(y3 * y3, axis=0)

    in_specs = [
        pl.BlockSpec((B, 1, NN, Cin), lambda t, j: (0, t + j, 0, 0)),
        pl.BlockSpec((1, Cin, Cout), lambda t, j: (j, 0, 0)),
        pl.BlockSpec((1, Cin, Cout), lambda t, j: (j, 0, 0)),
        pl.BlockSpec((1, Cin, Cout), lambda t, j: (j, 0, 0)),
        pl.BlockSpec((1, Cout), lambda t, j: (0, 0)),
        pl.BlockSpec((1, Cout), lambda t, j: (0, 0)),
        pl.BlockSpec((1, Cout), lambda t, j: (0, 0)),
    ]
    if has_norm:
        in_specs += [pl.BlockSpec((NN, 1), lambda t, j: (0, 0)),
                     pl.BlockSpec((NN, 1), lambda t, j: (0, 0))]
    out_specs = pl.BlockSpec((B, 1, NN, Cout), lambda t, j: (0, t, 0, 0))
    out_shape = jax.ShapeDtypeStruct((B, T_out, NN, Cout), _F32)
    if has_stats:
        out_specs = [out_specs, pl.BlockSpec((2, NN, Cout), lambda t, j: (0, 0, 0))]
        out_shape = [out_shape, jax.ShapeDtypeStruct((2, NN, Cout), _F32)]

    return functools.partial(
        _pcall,
        body,
        grid=(T_out, k),
        in_specs=in_specs,
        out_specs=out_specs,
        out_shape=out_shape,
        scratch_shapes=[pltpu.VMEM((BN, Cout), _F32)] * 3,
    )


def _tconv(x, p, norm, stats):
    B, T_in, _, Cin = x.shape
    Cout, _, k = p["Wp"].shape
    call = _make_tconv(B, T_in, Cin, Cout, k, norm is not None, stats)()
    args = [x,
            jnp.transpose(p["Wp"], (2, 1, 0)),
            jnp.transpose(p["Wq"], (2, 1, 0)),
            jnp.transpose(p["Wr"], (2, 1, 0)),
            p["bp"].reshape(1, Cout),
            p["bq"].reshape(1, Cout),
            p["br"].reshape(1, Cout)]
    if norm is not None:
        args += [norm[0], norm[1]]
    return call(*args)


def _make_cheb(BT, C):
    def body(x_ref, a_ref, m2_ref, w_ref, b_ref, out_ref):
        x = x_ref[0]
        t1 = jnp.dot(a_ref[...], x, preferred_element_type=_F32,
                     precision=lax.Precision.HIGHEST)
        t2 = jnp.dot(m2_ref[...], x, preferred_element_type=_F32,
                     precision=lax.Precision.HIGHEST)
        y = (_bdot(x, w_ref[0]) + _bdot(t1, w_ref[1]) + _bdot(t2, w_ref[2])
             + b_ref[...])
        out_ref[0] = jnp.maximum(y, 0.0)

    return functools.partial(
        _pcall,
        body,
        grid=(BT,),
        in_specs=[
            pl.BlockSpec((1, NN, C), lambda i: (i, 0, 0)),
            pl.BlockSpec((NN, NN), lambda i: (0, 0)),
            pl.BlockSpec((NN, NN), lambda i: (0, 0)),
            pl.BlockSpec((3, C, C), lambda i: (0, 0, 0)),
            pl.BlockSpec((1, C), lambda i: (0, 0)),
        ],
        out_specs=pl.BlockSpec((1, NN, C), lambda i: (i, 0, 0)),
        out_shape=jax.ShapeDtypeStruct((BT, NN, C), _F32),
    )


def _cheb(x, a, m2, w, b):
    B, T, _, C = x.shape
    call = _make_cheb(B * T, C)()
    y = call(x.reshape(B * T, NN, C), a, m2, w, b.reshape(1, C))
    return y.reshape(B, T, NN, C)


def _make_bnfinal(cnt):
    inv_cnt = 1.0 / float(cnt)

    def body(stats_ref, g_ref, b_ref, scale_ref, shift_ref):
        mean = jnp.sum(stats_ref[0], axis=1, keepdims=True) * inv_cnt
        var = jnp.sum(stats_ref[1], axis=1, keepdims=True) * inv_cnt - mean * mean
        inv = lax.rsqrt(var + 1e-5)
        scale = g_ref[...] * inv
        scale_ref[...] = scale
        shift_ref[...] = b_ref[...] - mean * scale

    return functools.partial(
        _pcall,
        body,
        out_shape=[jax.ShapeDtypeStruct((NN, 1), _F32),
                   jax.ShapeDtypeStruct((NN, 1), _F32)],
    )


def _bnfinal(stats, g, b, cnt):
    return _make_bnfinal(cnt)()(stats, g.reshape(NN, 1), b.reshape(NN, 1))


def _make_head(B, T, C):
    def body(x_ref, scale_ref, shift_ref, w1t_ref, b1_ref, w2t_ref, b2_ref,
             out_ref):
        x = x_ref[0]
        xn = x * scale_ref[...][None] + shift_ref[...][None]
        y1 = _bdot(xn.reshape(T * NN, C), w1t_ref[...]) + b1_ref[...]
        y2 = _bdot(y1, w2t_ref[...]) + b2_ref[...]
        out_ref[0] = y2.reshape(T, NN, 7)

    return functools.partial(
        _pcall,
        body,
        grid=(B,),
        in_specs=[
            pl.BlockSpec((1, T, NN, C), lambda b: (b, 0, 0, 0)),
            pl.BlockSpec((NN, 1), lambda b: (0, 0)),
            pl.BlockSpec((NN, 1), lambda b: (0, 0)),
            pl.BlockSpec((C, 32), lambda b: (0, 0)),
            pl.BlockSpec((1, 32), lambda b: (0, 0)),
            pl.BlockSpec((32, 7), lambda b: (0, 0)),
            pl.BlockSpec((1, 7), lambda b: (0, 0)),
        ],
        out_specs=pl.BlockSpec((1, T, NN, 7), lambda b: (b, 0, 0, 0)),
        out_shape=jax.ShapeDtypeStruct((B, T, NN, 7), _F32),
    )


def kernel(x, edge_index, edge_attr, params):
    row, col = edge_index[0], edge_index[1]
    pad = 16 * _EPT - row.shape[0]
    s_flat, deg = _build_a_sc(jnp.pad(row, (0, pad)),
                              jnp.pad(col, (0, pad)),
                              jnp.pad(edge_attr, (0, pad)))
    a, m2 = _finish_a(s_flat.reshape(_SROWS, _SROW),
                      deg[:NN].reshape(NN, 1),
                      deg[:_SROW].reshape(1, _SROW))

    t = x
    norm = None
    for bname in ("b1", "b2", "b3"):
        p = params[bname]
        t = _tconv(t, p["tc1"], norm=norm, stats=False)
        t = _cheb(t, a, m2, p["cheb_W"], p["cheb_b"])
        t, stats = _tconv(t, p["tc2"], norm=None, stats=True)
        cnt = t.shape[0] * t.shape[1] * t.shape[3]
        norm = _bnfinal(stats, p["bn_g"], p["bn_b"], cnt)

    B, T, _, C = t.shape
    head = _make_head(B, T, C)()
    return head(t, norm[0], norm[1],
                jnp.transpose(params["fc1_W"]),
                params["fc1_b"].reshape(1, 32),
                jnp.transpose(params["fc2_W"]),
                params["fc2_b"].reshape(1, 7))


# speedup vs baseline: 15.8805x; 1.2687x over previous
import functools

import jax
import jax.numpy as jnp
from jax import lax
from jax.experimental import pallas as pl
from jax.experimental.pallas import tpu as pltpu
from jax.experimental.pallas import tpu_sc as plsc

NN = 750
NP = 768
E_CH = 512
N_CHUNKS = 47

_F32 = jnp.float32
_BF16 = jnp.bfloat16


def _pcall(*args, **kwargs):
    return pl.pallas_call(*args, **kwargs)


def _bdot(a, b):
    return jnp.dot(a.astype(_BF16), b.astype(_BF16),
                   preferred_element_type=_F32)


def _build_a_body(row_ref, col_ref, attr_ref, out_ref, s_acc):
    i = pl.program_id(0)

    @pl.when(i == 0)
    def _():
        s_acc[...] = jnp.zeros_like(s_acc)

    r = row_ref[0]
    c = col_ref[0]
    a = attr_ref[0]
    iota_n = lax.broadcasted_iota(jnp.int32, (E_CH, NP), 1)
    ohr = jnp.where(iota_n == r, 1.0, 0.0).astype(_F32)
    ohc = jnp.where(iota_n == c, 1.0, 0.0).astype(_F32)
    s_acc[...] += lax.dot_general(
        ohr * a, ohc, (((0,), (0,)), ((), ())), preferred_element_type=_F32, precision=lax.Precision.HIGHEST)

    @pl.when(i == N_CHUNKS - 1)
    def _():
        s = s_acc[...]
        deg = jnp.sum(s, axis=1, keepdims=True)
        dis = jnp.where(deg > 0, lax.rsqrt(jnp.where(deg > 0, deg, 1.0)), 0.0)
        row_scaled = -(dis * s)
        eq = (lax.broadcasted_iota(jnp.int32, (NP, NP), 0)
              == lax.broadcasted_iota(jnp.int32, (NP, NP), 1))
        diag = jnp.where(eq, dis, 0.0)
        a_full = jnp.dot(row_scaled, diag, preferred_element_type=_F32, precision=lax.Precision.HIGHEST)
        out_ref[...] = a_full[:NN, :NN]


def _build_a(row3, col3, attr3):
    espec = pl.BlockSpec((1, E_CH, 1), lambda i: (i, 0, 0))
    return _pcall(
        _build_a_body,
        grid=(N_CHUNKS,),
        in_specs=[espec, espec, espec],
        out_specs=pl.BlockSpec((NN, NN), lambda i: (0, 0)),
        out_shape=jax.ShapeDtypeStruct((NN, NN), _F32),
        scratch_shapes=[pltpu.VMEM((NP, NP), _F32)],
    )(row3, col3, attr3)


_EPT = 1504
_NVR = _EPT // 16
_ROWS_PT = 48
_SROW = 752
_SROWS = 768


def _sc_build_body(row_hbm, col_hbm, attr_hbm, s_out, deg_out,
                   rowv, colv, attrv, idx2d, idxd, val2d, zbuf, sbuf,
                   s_shf, deg_sh):
    cid = lax.axis_index("c")
    sid = lax.axis_index("s")
    r0 = sid * _ROWS_PT

    @pl.when(cid == 0)
    def _phase_zero():
        for q in range(48):
            zbuf[pl.ds(q * 16, 16)] = jnp.zeros((16,), _F32)
        pltpu.sync_copy(zbuf.at[pl.ds(0, _ROWS_PT)],
                        deg_sh.at[pl.ds(r0, _ROWS_PT)])

        def zrow(i, _):
            r = r0 + i
            pltpu.sync_copy(zbuf.at[pl.ds(0, _SROW)],
                            s_shf.at[pl.ds(r * _SROW, _SROW)])
            return 0
        lax.fori_loop(0, _ROWS_PT, zrow, 0)

    plsc.subcore_barrier()

    @pl.when(cid == 0)
    def _phase_scatter():
        base = sid * _EPT
        pltpu.sync_copy(row_hbm.at[pl.ds(base, _EPT)], rowv)
        pltpu.sync_copy(col_hbm.at[pl.ds(base, _EPT)], colv)
        pltpu.sync_copy(attr_hbm.at[pl.ds(base, _EPT)], attrv)
        for q in range(2):
            idx2d[11, pl.ds(96 + q * 16, 16)] = jnp.zeros((16,), jnp.int32)
            idxd[11, pl.ds(96 + q * 16, 16)] = jnp.zeros((16,), jnp.int32)
            val2d[11, pl.ds(96 + q * 16, 16)] = jnp.zeros((16,), _F32)
        for v in range(_NVR):
            r16 = rowv[pl.ds(v * 16, 16)]
            c16 = colv[pl.ds(v * 16, 16)]
            a16 = attrv[pl.ds(v * 16, 16)]
            jr, jc = v // 8, (v % 8) * 16
            idx2d[jr, pl.ds(jc, 16)] = r16 * _SROW + c16
            idxd[jr, pl.ds(jc, 16)] = r16
            val2d[jr, pl.ds(jc, 16)] = a16
        for jrow in range(12):
            pltpu.sync_copy(val2d.at[jrow], s_shf.at[idx2d.at[jrow]], add=True)
        for jrow in range(12):
            pltpu.sync_copy(val2d.at[jrow], deg_sh.at[idxd.at[jrow]], add=True)

    plsc.subcore_barrier()

    @pl.when(cid == 0)
    def _phase_out():
        pltpu.sync_copy(deg_sh.at[pl.ds(r0, _ROWS_PT)],
                        zbuf.at[pl.ds(0, _ROWS_PT)])
        pltpu.sync_copy(zbuf.at[pl.ds(0, _ROWS_PT)],
                        deg_out.at[pl.ds(r0, _ROWS_PT)])
        blk = 8 * _SROW
        for k in range(_ROWS_PT // 8):
            off = r0 * _SROW + k * blk
            pltpu.sync_copy(s_shf.at[pl.ds(off, blk)], sbuf)
            pltpu.sync_copy(sbuf, s_out.at[pl.ds(off, blk)])


def _build_a_sc(row_p, col_p, attr_p):
    mesh = plsc.VectorSubcoreMesh(core_axis_name="c", subcore_axis_name="s")
    f = functools.partial(
        pl.kernel,
        mesh=mesh,
        out_type=[jax.ShapeDtypeStruct((_SROWS * _SROW,), _F32),
                  jax.ShapeDtypeStruct((_SROWS,), _F32)],
        scratch_types=[
            pltpu.VMEM((_EPT,), jnp.int32),
            pltpu.VMEM((_EPT,), jnp.int32),
            pltpu.VMEM((_EPT,), _F32),
            pltpu.VMEM((12, 128), jnp.int32),
            pltpu.VMEM((12, 128), jnp.int32),
            pltpu.VMEM((12, 128), _F32),
            pltpu.VMEM((768,), _F32),
            pltpu.VMEM((8 * _SROW,), _F32),
            pltpu.VMEM_SHARED((_SROWS * _SROW,), _F32),
            pltpu.VMEM_SHARED((_SROWS,), _F32),
        ],
    )(_sc_build_body)
    return f(row_p, col_p, attr_p)


def _finish_body(s_ref, degc_ref, degr_ref, a_ref, m2_ref):
    s = s_ref[...][:NN, :NN]
    dc = degc_ref[...]
    dr = degr_ref[...][:, :NN]
    disc = jnp.where(dc > 0, lax.rsqrt(jnp.where(dc > 0, dc, 1.0)), 0.0)
    disr = jnp.where(dr > 0, lax.rsqrt(jnp.where(dr > 0, dr, 1.0)), 0.0)
    a = -(disc * s * disr)
    a_ref[...] = a
    eye = jnp.where(
        lax.broadcasted_iota(jnp.int32, (NN, NN), 0)
        == lax.broadcasted_iota(jnp.int32, (NN, NN), 1), 1.0, 0.0)
    m2_ref[...] = 2.0 * jnp.dot(a, a, preferred_element_type=_F32, precision=lax.Precision.HIGHEST) - eye


def _finish_a(s2d, degc, degr):
    return _pcall(
        _finish_body,
        out_shape=[jax.ShapeDtypeStruct((NN, NN), _F32),
                   jax.ShapeDtypeStruct((NN, NN), _F32)],
    )(s2d, degc, degr)


def _make_tconv(B, T_in, Cin, Cout, k, has_norm, has_stats):
    T_out = T_in - k + 1
    BN = B * NN

    def body(*refs):
        x_ref, w_ref, bp_ref, bq_ref, br_ref = refs[:5]
        pos = 5
        if has_norm:
            scale_ref, shift_ref = refs[pos:pos + 2]
            pos += 2
        out_ref = refs[pos]
        pos += 1
        if has_stats:
            stats_ref = refs[pos]
            pos += 1
        ring = refs[pos]

        t = pl.program_id(0)

        if has_stats:
            @pl.when(t == 0)
            def _():
                stats_ref[...] = jnp.zeros_like(stats_ref)

        x = x_ref[...]
        if has_norm:
            x = x * scale_ref[...][None, None] + shift_ref[...][None, None]
        ring[t % k] = x.reshape(BN, Cin).astype(_BF16)

        @pl.when(t >= k - 1)
        def _():
            t0 = t - (k - 1)
            p = jnp.zeros((BN, Cout), _F32)
            q = jnp.zeros((BN, Cout), _F32)
            r = jnp.zeros((BN, Cout), _F32)
            for j in range(k):
                xj = ring[(t0 + j) % k]
                w = w_ref[j].astype(_BF16)
                p += jnp.dot(xj, w[0], preferred_element_type=_F32)
                q += jnp.dot(xj, w[1], preferred_element_type=_F32)
                r += jnp.dot(xj, w[2], preferred_element_type=_F32)
            p = p + bp_ref[...]
            q = jax.nn.sigmoid(q + bq_ref[...])
            r = r + br_ref[...]
            y = jnp.maximum(p * q + r, 0.0)
            out_ref[...] = y.reshape(B, 1, NN, Cout)
            if has_stats:
                y3 = y.reshape(B, NN, Cout)
                stats_ref[0] += jnp.sum(y3, axis=0)
                stats_ref[1] += jnp.sum(y3 * y3, axis=0)

    in_specs = [
        pl.BlockSpec((B, 1, NN, Cin), lambda t: (0, t, 0, 0)),
        pl.BlockSpec((k, 3, Cin, Cout), lambda t: (0, 0, 0, 0)),
        pl.BlockSpec((1, Cout), lambda t: (0, 0)),
        pl.BlockSpec((1, Cout), lambda t: (0, 0)),
        pl.BlockSpec((1, Cout), lambda t: (0, 0)),
    ]
    if has_norm:
        in_specs += [pl.BlockSpec((NN, 1), lambda t: (0, 0)),
                     pl.BlockSpec((NN, 1), lambda t: (0, 0))]
    out_specs = pl.BlockSpec(
        (B, 1, NN, Cout),
        lambda t: (0, jnp.maximum(t - (k - 1), 0), 0, 0))
    out_shape = jax.ShapeDtypeStruct((B, T_out, NN, Cout), _F32)
    if has_stats:
        out_specs = [out_specs, pl.BlockSpec((2, NN, Cout), lambda t: (0, 0, 0))]
        out_shape = [out_shape, jax.ShapeDtypeStruct((2, NN, Cout), _F32)]

    return functools.partial(
        _pcall,
        body,
        grid=(T_in,),
        in_specs=in_specs,
        out_specs=out_specs,
        out_shape=out_shape,
        scratch_shapes=[pltpu.VMEM((k, BN, Cin), _BF16)],
    )


def _tconv(x, p, norm, stats):
    B, T_in, _, Cin = x.shape
    Cout, _, k = p["Wp"].shape
    call = _make_tconv(B, T_in, Cin, Cout, k, norm is not None, stats)()
    w = jnp.stack([jnp.transpose(p["Wp"], (2, 1, 0)),
                   jnp.transpose(p["Wq"], (2, 1, 0)),
                   jnp.transpose(p["Wr"], (2, 1, 0))], axis=1)
    args = [x, w,
            p["bp"].reshape(1, Cout),
            p["bq"].reshape(1, Cout),
            p["br"].reshape(1, Cout)]
    if norm is not None:
        args += [norm[0], norm[1]]
    return call(*args)


def _make_cheb(BT, C):
    def body(x_ref, a_ref, m2_ref, w_ref, b_ref, out_ref):
        x = x_ref[0]
        t1 = jnp.dot(a_ref[...], x, preferred_element_type=_F32,
                     precision=lax.Precision.HIGHEST)
        t2 = jnp.dot(m2_ref[...], x, preferred_element_type=_F32,
                     precision=lax.Precision.HIGHEST)
        y = (_bdot(x, w_ref[0]) + _bdot(t1, w_ref[1]) + _bdot(t2, w_ref[2])
             + b_ref[...])
        out_ref[0] = jnp.maximum(y, 0.0)

    return functools.partial(
        _pcall,
        body,
        grid=(BT,),
        in_specs=[
            pl.BlockSpec((1, NN, C), lambda i: (i, 0, 0)),
            pl.BlockSpec((NN, NN), lambda i: (0, 0)),
            pl.BlockSpec((NN, NN), lambda i: (0, 0)),
            pl.BlockSpec((3, C, C), lambda i: (0, 0, 0)),
            pl.BlockSpec((1, C), lambda i: (0, 0)),
        ],
        out_specs=pl.BlockSpec((1, NN, C), lambda i: (i, 0, 0)),
        out_shape=jax.ShapeDtypeStruct((BT, NN, C), _F32),
    )


def _cheb(x, a, m2, w, b):
    B, T, _, C = x.shape
    call = _make_cheb(B * T, C)()
    y = call(x.reshape(B * T, NN, C), a, m2, w, b.reshape(1, C))
    return y.reshape(B, T, NN, C)


def _make_bnfinal(cnt):
    inv_cnt = 1.0 / float(cnt)

    def body(stats_ref, g_ref, b_ref, scale_ref, shift_ref):
        mean = jnp.sum(stats_ref[0], axis=1, keepdims=True) * inv_cnt
        var = jnp.sum(stats_ref[1], axis=1, keepdims=True) * inv_cnt - mean * mean
        inv = lax.rsqrt(var + 1e-5)
        scale = g_ref[...] * inv
        scale_ref[...] = scale
        shift_ref[...] = b_ref[...] - mean * scale

    return functools.partial(
        _pcall,
        body,
        out_shape=[jax.ShapeDtypeStruct((NN, 1), _F32),
                   jax.ShapeDtypeStruct((NN, 1), _F32)],
    )


def _bnfinal(stats, g, b, cnt):
    return _make_bnfinal(cnt)()(stats, g.reshape(NN, 1), b.reshape(NN, 1))


def _make_head(B, T, C):
    def body(x_ref, scale_ref, shift_ref, w1t_ref, b1_ref, w2t_ref, b2_ref,
             out_ref):
        x = x_ref[0]
        xn = x * scale_ref[...][None] + shift_ref[...][None]
        y1 = _bdot(xn.reshape(T * NN, C), w1t_ref[...]) + b1_ref[...]
        y2 = _bdot(y1, w2t_ref[...]) + b2_ref[...]
        out_ref[0] = y2.reshape(T, NN, 7)

    return functools.partial(
        _pcall,
        body,
        grid=(B,),
        in_specs=[
            pl.BlockSpec((1, T, NN, C), lambda b: (b, 0, 0, 0)),
            pl.BlockSpec((NN, 1), lambda b: (0, 0)),
            pl.BlockSpec((NN, 1), lambda b: (0, 0)),
            pl.BlockSpec((C, 32), lambda b: (0, 0)),
            pl.BlockSpec((1, 32), lambda b: (0, 0)),
            pl.BlockSpec((32, 7), lambda b: (0, 0)),
            pl.BlockSpec((1, 7), lambda b: (0, 0)),
        ],
        out_specs=pl.BlockSpec((1, T, NN, 7), lambda b: (b, 0, 0, 0)),
        out_shape=jax.ShapeDtypeStruct((B, T, NN, 7), _F32),
    )


def kernel(x, edge_index, edge_attr, params):
    row, col = edge_index[0], edge_index[1]
    pad = 16 * _EPT - row.shape[0]
    s_flat, deg = _build_a_sc(jnp.pad(row, (0, pad)),
                              jnp.pad(col, (0, pad)),
                              jnp.pad(edge_attr, (0, pad)))
    a, m2 = _finish_a(s_flat.reshape(_SROWS, _SROW),
                      deg[:NN].reshape(NN, 1),
                      deg[:_SROW].reshape(1, _SROW))

    t = x
    norm = None
    for bname in ("b1", "b2", "b3"):
        p = params[bname]
        t = _tconv(t, p["tc1"], norm=norm, stats=False)
        t = _cheb(t, a, m2, p["cheb_W"], p["cheb_b"])
        t, stats = _tconv(t, p["tc2"], norm=None, stats=True)
        cnt = t.shape[0] * t.shape[1] * t.shape[3]
        norm = _bnfinal(stats, p["bn_g"], p["bn_b"], cnt)

    B, T, _, C = t.shape
    head = _make_head(B, T, C)()
    return head(t, norm[0], norm[1],
                jnp.transpose(params["fc1_W"]),
                params["fc1_b"].reshape(1, 32),
                jnp.transpose(params["fc2_W"]),
                params["fc2_b"].reshape(1, 7))


# speedup vs baseline: 20.3528x; 1.2816x over previous
import functools

import jax
import jax.numpy as jnp
from jax import lax
from jax.experimental import pallas as pl
from jax.experimental.pallas import tpu as pltpu
from jax.experimental.pallas import tpu_sc as plsc

NN = 750
NP = 768
E_CH = 512
N_CHUNKS = 47

_F32 = jnp.float32
_BF16 = jnp.bfloat16


def _pcall(*args, **kwargs):
    return pl.pallas_call(*args, **kwargs)


def _bdot(a, b):
    return jnp.dot(a.astype(_BF16), b.astype(_BF16),
                   preferred_element_type=_F32)


def _build_a_body(row_ref, col_ref, attr_ref, out_ref, s_acc):
    i = pl.program_id(0)

    @pl.when(i == 0)
    def _():
        s_acc[...] = jnp.zeros_like(s_acc)

    r = row_ref[0]
    c = col_ref[0]
    a = attr_ref[0]
    iota_n = lax.broadcasted_iota(jnp.int32, (E_CH, NP), 1)
    ohr = jnp.where(iota_n == r, 1.0, 0.0).astype(_F32)
    ohc = jnp.where(iota_n == c, 1.0, 0.0).astype(_F32)
    s_acc[...] += lax.dot_general(
        ohr * a, ohc, (((0,), (0,)), ((), ())), preferred_element_type=_F32, precision=lax.Precision.HIGHEST)

    @pl.when(i == N_CHUNKS - 1)
    def _():
        s = s_acc[...]
        deg = jnp.sum(s, axis=1, keepdims=True)
        dis = jnp.where(deg > 0, lax.rsqrt(jnp.where(deg > 0, deg, 1.0)), 0.0)
        row_scaled = -(dis * s)
        eq = (lax.broadcasted_iota(jnp.int32, (NP, NP), 0)
              == lax.broadcasted_iota(jnp.int32, (NP, NP), 1))
        diag = jnp.where(eq, dis, 0.0)
        a_full = jnp.dot(row_scaled, diag, preferred_element_type=_F32, precision=lax.Precision.HIGHEST)
        out_ref[...] = a_full[:NN, :NN]


def _build_a(row3, col3, attr3):
    espec = pl.BlockSpec((1, E_CH, 1), lambda i: (i, 0, 0))
    return _pcall(
        _build_a_body,
        grid=(N_CHUNKS,),
        in_specs=[espec, espec, espec],
        out_specs=pl.BlockSpec((NN, NN), lambda i: (0, 0)),
        out_shape=jax.ShapeDtypeStruct((NN, NN), _F32),
        scratch_shapes=[pltpu.VMEM((NP, NP), _F32)],
    )(row3, col3, attr3)


_EPT = 1504
_NVR = _EPT // 16
_ROWS_PT = 48
_SROW = 752
_SROWS = 768


def _sc_build_body(row_hbm, col_hbm, attr_hbm, s_out, deg_out,
                   rowv, colv, attrv, idx2d, idxd, val2d, zbuf, sbuf,
                   s_shf, deg_sh):
    cid = lax.axis_index("c")
    sid = lax.axis_index("s")
    r0 = sid * _ROWS_PT

    @pl.when(cid == 0)
    def _phase_zero():
        for q in range(48):
            zbuf[pl.ds(q * 16, 16)] = jnp.zeros((16,), _F32)
        pltpu.sync_copy(zbuf.at[pl.ds(0, _ROWS_PT)],
                        deg_sh.at[pl.ds(r0, _ROWS_PT)])

        def zrow(i, _):
            r = r0 + i
            pltpu.sync_copy(zbuf.at[pl.ds(0, _SROW)],
                            s_shf.at[pl.ds(r * _SROW, _SROW)])
            return 0
        lax.fori_loop(0, _ROWS_PT, zrow, 0)

    plsc.subcore_barrier()

    @pl.when(cid == 0)
    def _phase_scatter():
        base = sid * _EPT
        pltpu.sync_copy(row_hbm.at[pl.ds(base, _EPT)], rowv)
        pltpu.sync_copy(col_hbm.at[pl.ds(base, _EPT)], colv)
        pltpu.sync_copy(attr_hbm.at[pl.ds(base, _EPT)], attrv)
        for q in range(2):
            idx2d[11, pl.ds(96 + q * 16, 16)] = jnp.zeros((16,), jnp.int32)
            idxd[11, pl.ds(96 + q * 16, 16)] = jnp.zeros((16,), jnp.int32)
            val2d[11, pl.ds(96 + q * 16, 16)] = jnp.zeros((16,), _F32)
        for v in range(_NVR):
            r16 = rowv[pl.ds(v * 16, 16)]
            c16 = colv[pl.ds(v * 16, 16)]
            a16 = attrv[pl.ds(v * 16, 16)]
            jr, jc = v // 8, (v % 8) * 16
            idx2d[jr, pl.ds(jc, 16)] = r16 * _SROW + c16
            idxd[jr, pl.ds(jc, 16)] = r16
            val2d[jr, pl.ds(jc, 16)] = a16
        for jrow in range(12):
            pltpu.sync_copy(val2d.at[jrow], s_shf.at[idx2d.at[jrow]], add=True)
        for jrow in range(12):
            pltpu.sync_copy(val2d.at[jrow], deg_sh.at[idxd.at[jrow]], add=True)

    plsc.subcore_barrier()

    @pl.when(cid == 0)
    def _phase_out():
        pltpu.sync_copy(deg_sh.at[pl.ds(r0, _ROWS_PT)],
                        zbuf.at[pl.ds(0, _ROWS_PT)])
        pltpu.sync_copy(zbuf.at[pl.ds(0, _ROWS_PT)],
                        deg_out.at[pl.ds(r0, _ROWS_PT)])
        blk = 8 * _SROW
        for k in range(_ROWS_PT // 8):
            off = r0 * _SROW + k * blk
            pltpu.sync_copy(s_shf.at[pl.ds(off, blk)], sbuf)
            pltpu.sync_copy(sbuf, s_out.at[pl.ds(off, blk)])


def _build_a_sc(row_p, col_p, attr_p):
    mesh = plsc.VectorSubcoreMesh(core_axis_name="c", subcore_axis_name="s")
    f = functools.partial(
        pl.kernel,
        mesh=mesh,
        out_type=[jax.ShapeDtypeStruct((_SROWS * _SROW,), _F32),
                  jax.ShapeDtypeStruct((_SROWS,), _F32)],
        scratch_types=[
            pltpu.VMEM((_EPT,), jnp.int32),
            pltpu.VMEM((_EPT,), jnp.int32),
            pltpu.VMEM((_EPT,), _F32),
            pltpu.VMEM((12, 128), jnp.int32),
            pltpu.VMEM((12, 128), jnp.int32),
            pltpu.VMEM((12, 128), _F32),
            pltpu.VMEM((768,), _F32),
            pltpu.VMEM((8 * _SROW,), _F32),
            pltpu.VMEM_SHARED((_SROWS * _SROW,), _F32),
            pltpu.VMEM_SHARED((_SROWS,), _F32),
        ],
    )(_sc_build_body)
    return f(row_p, col_p, attr_p)


def _finish_body(s_ref, degc_ref, degr_ref, a_ref, m2_ref):
    s = s_ref[...][:NN, :NN]
    dc = degc_ref[...]
    dr = degr_ref[...][:, :NN]
    disc = jnp.where(dc > 0, lax.rsqrt(jnp.where(dc > 0, dc, 1.0)), 0.0)
    disr = jnp.where(dr > 0, lax.rsqrt(jnp.where(dr > 0, dr, 1.0)), 0.0)
    a = -(disc * s * disr)
    a_ref[...] = a
    eye = jnp.where(
        lax.broadcasted_iota(jnp.int32, (NN, NN), 0)
        == lax.broadcasted_iota(jnp.int32, (NN, NN), 1), 1.0, 0.0)
    m2_ref[...] = 2.0 * jnp.dot(a, a, preferred_element_type=_F32, precision=lax.Precision.HIGHEST) - eye


def _finish_a(s2d, degc, degr):
    return _pcall(
        _finish_body,
        out_shape=[jax.ShapeDtypeStruct((NN, NN), _F32),
                   jax.ShapeDtypeStruct((NN, NN), _F32)],
    )(s2d, degc, degr)


def _make_tconv(B, T_in, Cin, Cout, k, has_norm, has_stats):
    T_out = T_in - k + 1
    BN = B * NN

    def body(*refs):
        x_ref, w_ref, bp_ref, bq_ref, br_ref = refs[:5]
        pos = 5
        if has_norm:
            scale_ref, shift_ref = refs[pos:pos + 2]
            pos += 2
        out_ref = refs[pos]
        pos += 1
        if has_stats:
            stats_ref = refs[pos]
            pos += 1
        ring = refs[pos]

        t = pl.program_id(0)

        if has_stats:
            @pl.when(t == 0)
            def _():
                stats_ref[...] = jnp.zeros_like(stats_ref)

        x = x_ref[...]
        if has_norm:
            x = x * scale_ref[...][None, None] + shift_ref[...][None, None]
        ring[t % k] = x.reshape(BN, Cin).astype(_BF16)

        @pl.when(t >= k - 1)
        def _():
            t0 = t - (k - 1)
            p = jnp.zeros((BN, Cout), _F32)
            q = jnp.zeros((BN, Cout), _F32)
            r = jnp.zeros((BN, Cout), _F32)
            for j in range(k):
                xj = ring[(t0 + j) % k]
                w = w_ref[j].astype(_BF16)
                p += jnp.dot(xj, w[0], preferred_element_type=_F32)
                q += jnp.dot(xj, w[1], preferred_element_type=_F32)
                r += jnp.dot(xj, w[2], preferred_element_type=_F32)
            p = p + bp_ref[...]
            q = jax.nn.sigmoid(q + bq_ref[...])
            r = r + br_ref[...]
            y = jnp.maximum(p * q + r, 0.0)
            out_ref[...] = y.reshape(B, 1, NN, Cout)
            if has_stats:
                y3 = y.reshape(B, NN, Cout)
                stats_ref[0] += jnp.sum(y3, axis=0)
                stats_ref[1] += jnp.sum(y3 * y3, axis=0)

    in_specs = [
        pl.BlockSpec((B, 1, NN, Cin), lambda t: (0, t, 0, 0)),
        pl.BlockSpec((k, 3, Cin, Cout), lambda t: (0, 0, 0, 0)),
        pl.BlockSpec((1, Cout), lambda t: (0, 0)),
        pl.BlockSpec((1, Cout), lambda t: (0, 0)),
        pl.BlockSpec((1, Cout), lambda t: (0, 0)),
    ]
    if has_norm:
        in_specs += [pl.BlockSpec((NN, 1), lambda t: (0, 0)),
                     pl.BlockSpec((NN, 1), lambda t: (0, 0))]
    out_specs = pl.BlockSpec(
        (B, 1, NN, Cout),
        lambda t: (0, jnp.maximum(t - (k - 1), 0), 0, 0))
    out_shape = jax.ShapeDtypeStruct((B, T_out, NN, Cout), _F32)
    if has_stats:
        out_specs = [out_specs, pl.BlockSpec((2, NN, Cout), lambda t: (0, 0, 0))]
        out_shape = [out_shape, jax.ShapeDtypeStruct((2, NN, Cout), _F32)]

    return functools.partial(
        _pcall,
        body,
        grid=(T_in,),
        in_specs=in_specs,
        out_specs=out_specs,
        out_shape=out_shape,
        scratch_shapes=[pltpu.VMEM((k, BN, Cin), _BF16)],
    )


def _tconv(x, p, norm, stats):
    B, T_in, _, Cin = x.shape
    Cout, _, k = p["Wp"].shape
    call = _make_tconv(B, T_in, Cin, Cout, k, norm is not None, stats)()
    w = jnp.stack([jnp.transpose(p["Wp"], (2, 1, 0)),
                   jnp.transpose(p["Wq"], (2, 1, 0)),
                   jnp.transpose(p["Wr"], (2, 1, 0))], axis=1)
    args = [x, w,
            p["bp"].reshape(1, Cout),
            p["bq"].reshape(1, Cout),
            p["br"].reshape(1, Cout)]
    if norm is not None:
        args += [norm[0], norm[1]]
    return call(*args)


def _make_cheb(BT, C):
    def body(x_ref, a_ref, m2_ref, w_ref, b_ref, out_ref,
             ahi, alo, mhi, mlo):
        i = pl.program_id(0)

        @pl.when(i == 0)
        def _():
            a = a_ref[...]
            ah = a.astype(_BF16)
            ahi[...] = ah
            alo[...] = (a - ah.astype(_F32)).astype(_BF16)
            m2 = m2_ref[...]
            mh = m2.astype(_BF16)
            mhi[...] = mh
            mlo[...] = (m2 - mh.astype(_F32)).astype(_BF16)

        x = x_ref[0]
        xh = x.astype(_BF16)
        xl = (x - xh.astype(_F32)).astype(_BF16)
        t1 = (jnp.dot(ahi[...], xh, preferred_element_type=_F32)
              + jnp.dot(ahi[...], xl, preferred_element_type=_F32)
              + jnp.dot(alo[...], xh, preferred_element_type=_F32))
        t2 = (jnp.dot(mhi[...], xh, preferred_element_type=_F32)
              + jnp.dot(mhi[...], xl, preferred_element_type=_F32)
              + jnp.dot(mlo[...], xh, preferred_element_type=_F32))
        y = (_bdot(x, w_ref[0]) + _bdot(t1, w_ref[1]) + _bdot(t2, w_ref[2])
             + b_ref[...])
        out_ref[0] = jnp.maximum(y, 0.0)

    return functools.partial(
        _pcall,
        body,
        grid=(BT,),
        in_specs=[
            pl.BlockSpec((1, NN, C), lambda i: (i, 0, 0)),
            pl.BlockSpec((NN, NN), lambda i: (0, 0)),
            pl.BlockSpec((NN, NN), lambda i: (0, 0)),
            pl.BlockSpec((3, C, C), lambda i: (0, 0, 0)),
            pl.BlockSpec((1, C), lambda i: (0, 0)),
        ],
        out_specs=pl.BlockSpec((1, NN, C), lambda i: (i, 0, 0)),
        out_shape=jax.ShapeDtypeStruct((BT, NN, C), _F32),
        scratch_shapes=[pltpu.VMEM((NN, NN), _BF16) for _ in range(4)],
    )


def _cheb(x, a, m2, w, b):
    B, T, _, C = x.shape
    call = _make_cheb(B * T, C)()
    y = call(x.reshape(B * T, NN, C), a, m2, w, b.reshape(1, C))
    return y.reshape(B, T, NN, C)


def _make_bnfinal(cnt):
    inv_cnt = 1.0 / float(cnt)

    def body(stats_ref, g_ref, b_ref, scale_ref, shift_ref):
        mean = jnp.sum(stats_ref[0], axis=1, keepdims=True) * inv_cnt
        var = jnp.sum(stats_ref[1], axis=1, keepdims=True) * inv_cnt - mean * mean
        inv = lax.rsqrt(var + 1e-5)
        scale = g_ref[...] * inv
        scale_ref[...] = scale
        shift_ref[...] = b_ref[...] - mean * scale

    return functools.partial(
        _pcall,
        body,
        out_shape=[jax.ShapeDtypeStruct((NN, 1), _F32),
                   jax.ShapeDtypeStruct((NN, 1), _F32)],
    )


def _bnfinal(stats, g, b, cnt):
    return _make_bnfinal(cnt)()(stats, g.reshape(NN, 1), b.reshape(NN, 1))


def _make_head(B, T, C):
    def body(x_ref, scale_ref, shift_ref, w1t_ref, b1_ref, w2t_ref, b2_ref,
             out_ref):
        x = x_ref[0]
        xn = x * scale_ref[...][None] + shift_ref[...][None]
        y1 = _bdot(xn.reshape(T * NN, C), w1t_ref[...]) + b1_ref[...]
        y2 = _bdot(y1, w2t_ref[...]) + b2_ref[...]
        out_ref[0] = y2.reshape(T, NN, 7)

    return functools.partial(
        _pcall,
        body,
        grid=(B,),
        in_specs=[
            pl.BlockSpec((1, T, NN, C), lambda b: (b, 0, 0, 0)),
            pl.BlockSpec((NN, 1), lambda b: (0, 0)),
            pl.BlockSpec((NN, 1), lambda b: (0, 0)),
            pl.BlockSpec((C, 32), lambda b: (0, 0)),
            pl.BlockSpec((1, 32), lambda b: (0, 0)),
            pl.BlockSpec((32, 7), lambda b: (0, 0)),
            pl.BlockSpec((1, 7), lambda b: (0, 0)),
        ],
        out_specs=pl.BlockSpec((1, T, NN, 7), lambda b: (b, 0, 0, 0)),
        out_shape=jax.ShapeDtypeStruct((B, T, NN, 7), _F32),
    )


def kernel(x, edge_index, edge_attr, params):
    row, col = edge_index[0], edge_index[1]
    pad = 16 * _EPT - row.shape[0]
    s_flat, deg = _build_a_sc(jnp.pad(row, (0, pad)),
                              jnp.pad(col, (0, pad)),
                              jnp.pad(edge_attr, (0, pad)))
    a, m2 = _finish_a(s_flat.reshape(_SROWS, _SROW),
                      deg[:NN].reshape(NN, 1),
                      deg[:_SROW].reshape(1, _SROW))

    t = x
    norm = None
    for bname in ("b1", "b2", "b3"):
        p = params[bname]
        t = _tconv(t, p["tc1"], norm=norm, stats=False)
        t = _cheb(t, a, m2, p["cheb_W"], p["cheb_b"])
        t, stats = _tconv(t, p["tc2"], norm=None, stats=True)
        cnt = t.shape[0] * t.shape[1] * t.shape[3]
        norm = _bnfinal(stats, p["bn_g"], p["bn_b"], cnt)

    B, T, _, C = t.shape
    head = _make_head(B, T, C)()
    return head(t, norm[0], norm[1],
                jnp.transpose(params["fc1_W"]),
                params["fc1_b"].reshape(1, 32),
                jnp.transpose(params["fc2_W"]),
                params["fc2_b"].reshape(1, 7))


# speedup vs baseline: 24.0270x; 1.1805x over previous
import functools

import jax
import jax.numpy as jnp
from jax import lax
from jax.experimental import pallas as pl
from jax.experimental.pallas import tpu as pltpu
from jax.experimental.pallas import tpu_sc as plsc

NN = 750
NP = 768
E_CH = 512
N_CHUNKS = 47

_F32 = jnp.float32
_BF16 = jnp.bfloat16


def _pcall(*args, **kwargs):
    return pl.pallas_call(*args, **kwargs)


def _bdot(a, b):
    return jnp.dot(a.astype(_BF16), b.astype(_BF16),
                   preferred_element_type=_F32)


def _build_a_body(row_ref, col_ref, attr_ref, out_ref, s_acc):
    i = pl.program_id(0)

    @pl.when(i == 0)
    def _():
        s_acc[...] = jnp.zeros_like(s_acc)

    r = row_ref[0]
    c = col_ref[0]
    a = attr_ref[0]
    iota_n = lax.broadcasted_iota(jnp.int32, (E_CH, NP), 1)
    ohr = jnp.where(iota_n == r, 1.0, 0.0).astype(_F32)
    ohc = jnp.where(iota_n == c, 1.0, 0.0).astype(_F32)
    s_acc[...] += lax.dot_general(
        ohr * a, ohc, (((0,), (0,)), ((), ())), preferred_element_type=_F32, precision=lax.Precision.HIGHEST)

    @pl.when(i == N_CHUNKS - 1)
    def _():
        s = s_acc[...]
        deg = jnp.sum(s, axis=1, keepdims=True)
        dis = jnp.where(deg > 0, lax.rsqrt(jnp.where(deg > 0, deg, 1.0)), 0.0)
        row_scaled = -(dis * s)
        eq = (lax.broadcasted_iota(jnp.int32, (NP, NP), 0)
              == lax.broadcasted_iota(jnp.int32, (NP, NP), 1))
        diag = jnp.where(eq, dis, 0.0)
        a_full = jnp.dot(row_scaled, diag, preferred_element_type=_F32, precision=lax.Precision.HIGHEST)
        out_ref[...] = a_full[:NN, :NN]


def _build_a(row3, col3, attr3):
    espec = pl.BlockSpec((1, E_CH, 1), lambda i: (i, 0, 0))
    return _pcall(
        _build_a_body,
        grid=(N_CHUNKS,),
        in_specs=[espec, espec, espec],
        out_specs=pl.BlockSpec((NN, NN), lambda i: (0, 0)),
        out_shape=jax.ShapeDtypeStruct((NN, NN), _F32),
        scratch_shapes=[pltpu.VMEM((NP, NP), _F32)],
    )(row3, col3, attr3)


_EPT = 1504
_NVR = _EPT // 16
_ROWS_PT = 48
_SROW = 752
_SROWS = 768


def _sc_build_body(row_hbm, col_hbm, attr_hbm, s_out, deg_out,
                   rowv, colv, attrv, idx2d, idxd, val2d, zbuf, sbuf,
                   s_shf, deg_sh):
    cid = lax.axis_index("c")
    sid = lax.axis_index("s")
    r0 = sid * _ROWS_PT

    @pl.when(cid == 0)
    def _phase_zero():
        for q in range(48):
            zbuf[pl.ds(q * 16, 16)] = jnp.zeros((16,), _F32)
        pltpu.sync_copy(zbuf.at[pl.ds(0, _ROWS_PT)],
                        deg_sh.at[pl.ds(r0, _ROWS_PT)])

        def zrow(i, _):
            r = r0 + i
            pltpu.sync_copy(zbuf.at[pl.ds(0, _SROW)],
                            s_shf.at[pl.ds(r * _SROW, _SROW)])
            return 0
        lax.fori_loop(0, _ROWS_PT, zrow, 0)

    plsc.subcore_barrier()

    @pl.when(cid == 0)
    def _phase_scatter():
        base = sid * _EPT
        pltpu.sync_copy(row_hbm.at[pl.ds(base, _EPT)], rowv)
        pltpu.sync_copy(col_hbm.at[pl.ds(base, _EPT)], colv)
        pltpu.sync_copy(attr_hbm.at[pl.ds(base, _EPT)], attrv)
        for q in range(2):
            idx2d[11, pl.ds(96 + q * 16, 16)] = jnp.zeros((16,), jnp.int32)
            idxd[11, pl.ds(96 + q * 16, 16)] = jnp.zeros((16,), jnp.int32)
            val2d[11, pl.ds(96 + q * 16, 16)] = jnp.zeros((16,), _F32)
        for v in range(_NVR):
            r16 = rowv[pl.ds(v * 16, 16)]
            c16 = colv[pl.ds(v * 16, 16)]
            a16 = attrv[pl.ds(v * 16, 16)]
            jr, jc = v // 8, (v % 8) * 16
            idx2d[jr, pl.ds(jc, 16)] = r16 * _SROW + c16
            idxd[jr, pl.ds(jc, 16)] = r16
            val2d[jr, pl.ds(jc, 16)] = a16
        for jrow in range(12):
            pltpu.sync_copy(val2d.at[jrow], s_shf.at[idx2d.at[jrow]], add=True)
        for jrow in range(12):
            pltpu.sync_copy(val2d.at[jrow], deg_sh.at[idxd.at[jrow]], add=True)

    plsc.subcore_barrier()

    @pl.when(cid == 0)
    def _phase_out():
        pltpu.sync_copy(deg_sh.at[pl.ds(r0, _ROWS_PT)],
                        zbuf.at[pl.ds(0, _ROWS_PT)])
        pltpu.sync_copy(zbuf.at[pl.ds(0, _ROWS_PT)],
                        deg_out.at[pl.ds(r0, _ROWS_PT)])
        blk = 8 * _SROW
        for k in range(_ROWS_PT // 8):
            off = r0 * _SROW + k * blk
            pltpu.sync_copy(s_shf.at[pl.ds(off, blk)], sbuf)
            pltpu.sync_copy(sbuf, s_out.at[pl.ds(off, blk)])


def _build_a_sc(row_p, col_p, attr_p):
    mesh = plsc.VectorSubcoreMesh(core_axis_name="c", subcore_axis_name="s")
    f = functools.partial(
        pl.kernel,
        mesh=mesh,
        out_type=[jax.ShapeDtypeStruct((_SROWS * _SROW,), _F32),
                  jax.ShapeDtypeStruct((_SROWS,), _F32)],
        scratch_types=[
            pltpu.VMEM((_EPT,), jnp.int32),
            pltpu.VMEM((_EPT,), jnp.int32),
            pltpu.VMEM((_EPT,), _F32),
            pltpu.VMEM((12, 128), jnp.int32),
            pltpu.VMEM((12, 128), jnp.int32),
            pltpu.VMEM((12, 128), _F32),
            pltpu.VMEM((768,), _F32),
            pltpu.VMEM((8 * _SROW,), _F32),
            pltpu.VMEM_SHARED((_SROWS * _SROW,), _F32),
            pltpu.VMEM_SHARED((_SROWS,), _F32),
        ],
    )(_sc_build_body)
    return f(row_p, col_p, attr_p)


def _finish_body(s_ref, degc_ref, degr_ref, a_ref, m2_ref):
    s = s_ref[...][:NN, :NN]
    dc = degc_ref[...]
    dr = degr_ref[...][:, :NN]
    disc = jnp.where(dc > 0, lax.rsqrt(jnp.where(dc > 0, dc, 1.0)), 0.0)
    disr = jnp.where(dr > 0, lax.rsqrt(jnp.where(dr > 0, dr, 1.0)), 0.0)
    a = -(disc * s * disr)
    a_ref[...] = a
    eye = jnp.where(
        lax.broadcasted_iota(jnp.int32, (NN, NN), 0)
        == lax.broadcasted_iota(jnp.int32, (NN, NN), 1), 1.0, 0.0)
    m2_ref[...] = 2.0 * jnp.dot(a, a, preferred_element_type=_F32, precision=lax.Precision.HIGHEST) - eye


def _finish_a(s2d, degc, degr):
    return _pcall(
        _finish_body,
        out_shape=[jax.ShapeDtypeStruct((NN, NN), _F32),
                   jax.ShapeDtypeStruct((NN, NN), _F32)],
    )(s2d, degc, degr)


def _make_tconv(B, T_in, Cin, Cout, k, has_norm, has_stats):
    T_out = T_in - k + 1
    BN = B * NN

    def body(*refs):
        x_ref, w_ref, bp_ref, bq_ref, br_ref = refs[:5]
        pos = 5
        if has_norm:
            scale_ref, shift_ref = refs[pos:pos + 2]
            pos += 2
        out_ref = refs[pos]
        pos += 1
        if has_stats:
            stats_ref = refs[pos]
            pos += 1
        ring = refs[pos]

        t = pl.program_id(0)

        if has_stats:
            @pl.when(t == 0)
            def _():
                stats_ref[...] = jnp.zeros_like(stats_ref)

        x = x_ref[...]
        if has_norm:
            x = x * scale_ref[...][None, None] + shift_ref[...][None, None]
        ring[t % k] = x.reshape(BN, Cin).astype(_BF16)

        @pl.when(t >= k - 1)
        def _():
            t0 = t - (k - 1)
            p = jnp.zeros((BN, Cout), _F32)
            q = jnp.zeros((BN, Cout), _F32)
            r = jnp.zeros((BN, Cout), _F32)
            for j in range(k):
                xj = ring[(t0 + j) % k]
                w = w_ref[j].astype(_BF16)
                p += jnp.dot(xj, w[0], preferred_element_type=_F32)
                q += jnp.dot(xj, w[1], preferred_element_type=_F32)
                r += jnp.dot(xj, w[2], preferred_element_type=_F32)
            p = p + bp_ref[...]
            q = jax.nn.sigmoid(q + bq_ref[...])
            r = r + br_ref[...]
            y = jnp.maximum(p * q + r, 0.0)
            out_ref[...] = y.reshape(B, 1, NN, Cout)
            if has_stats:
                y3 = y.reshape(B, NN, Cout)
                stats_ref[0] += jnp.sum(y3, axis=0)
                stats_ref[1] += jnp.sum(y3 * y3, axis=0)

    in_specs = [
        pl.BlockSpec((B, 1, NN, Cin), lambda t: (0, t, 0, 0)),
        pl.BlockSpec((k, 3, Cin, Cout), lambda t: (0, 0, 0, 0)),
        pl.BlockSpec((1, Cout), lambda t: (0, 0)),
        pl.BlockSpec((1, Cout), lambda t: (0, 0)),
        pl.BlockSpec((1, Cout), lambda t: (0, 0)),
    ]
    if has_norm:
        in_specs += [pl.BlockSpec((NN, 1), lambda t: (0, 0)),
                     pl.BlockSpec((NN, 1), lambda t: (0, 0))]
    out_specs = pl.BlockSpec(
        (B, 1, NN, Cout),
        lambda t: (0, jnp.maximum(t - (k - 1), 0), 0, 0))
    out_shape = jax.ShapeDtypeStruct((B, T_out, NN, Cout), _F32)
    if has_stats:
        out_specs = [out_specs, pl.BlockSpec((2, NN, Cout), lambda t: (0, 0, 0))]
        out_shape = [out_shape, jax.ShapeDtypeStruct((2, NN, Cout), _F32)]

    return functools.partial(
        _pcall,
        body,
        grid=(T_in,),
        in_specs=in_specs,
        out_specs=out_specs,
        out_shape=out_shape,
        scratch_shapes=[pltpu.VMEM((k, BN, Cin), _BF16)],
    )


def _tconv(x, p, norm, stats):
    B, T_in, _, Cin = x.shape
    Cout, _, k = p["Wp"].shape
    call = _make_tconv(B, T_in, Cin, Cout, k, norm is not None, stats)()
    w = jnp.stack([jnp.transpose(p["Wp"], (2, 1, 0)),
                   jnp.transpose(p["Wq"], (2, 1, 0)),
                   jnp.transpose(p["Wr"], (2, 1, 0))], axis=1)
    args = [x, w,
            p["bp"].reshape(1, Cout),
            p["bq"].reshape(1, Cout),
            p["br"].reshape(1, Cout)]
    if norm is not None:
        args += [norm[0], norm[1]]
    return call(*args)


def _make_cheb(BT, C):
    def body(x_ref, a_ref, m2_ref, w_ref, b_ref, out_ref,
             ahi, alo, mhi, mlo):
        i = pl.program_id(0)

        @pl.when(i == 0)
        def _():
            a = a_ref[...]
            ah = a.astype(_BF16)
            ahi[...] = ah
            alo[...] = (a - ah.astype(_F32)).astype(_BF16)
            m2 = m2_ref[...]
            mh = m2.astype(_BF16)
            mhi[...] = mh
            mlo[...] = (m2 - mh.astype(_F32)).astype(_BF16)

        x = jnp.concatenate([x_ref[0], x_ref[1]], axis=1)
        xh = x.astype(_BF16)
        xl = (x - xh.astype(_F32)).astype(_BF16)
        t1 = (jnp.dot(ahi[...], xh, preferred_element_type=_F32)
              + jnp.dot(ahi[...], xl, preferred_element_type=_F32)
              + jnp.dot(alo[...], xh, preferred_element_type=_F32))
        t2 = (jnp.dot(mhi[...], xh, preferred_element_type=_F32)
              + jnp.dot(mhi[...], xl, preferred_element_type=_F32)
              + jnp.dot(mlo[...], xh, preferred_element_type=_F32))
        for j in range(2):
            lo, hi = j * C, (j + 1) * C
            y = (_bdot(x[:, lo:hi], w_ref[0]) + _bdot(t1[:, lo:hi], w_ref[1])
                 + _bdot(t2[:, lo:hi], w_ref[2]) + b_ref[...])
            out_ref[j] = jnp.maximum(y, 0.0)

    assert BT % 2 == 0
    return functools.partial(
        _pcall,
        body,
        grid=(BT // 2,),
        in_specs=[
            pl.BlockSpec((2, NN, C), lambda i: (i, 0, 0)),
            pl.BlockSpec((NN, NN), lambda i: (0, 0)),
            pl.BlockSpec((NN, NN), lambda i: (0, 0)),
            pl.BlockSpec((3, C, C), lambda i: (0, 0, 0)),
            pl.BlockSpec((1, C), lambda i: (0, 0)),
        ],
        out_specs=pl.BlockSpec((2, NN, C), lambda i: (i, 0, 0)),
        out_shape=jax.ShapeDtypeStruct((BT, NN, C), _F32),
        scratch_shapes=[pltpu.VMEM((NN, NN), _BF16) for _ in range(4)],
    )


def _cheb(x, a, m2, w, b):
    B, T, _, C = x.shape
    call = _make_cheb(B * T, C)()
    y = call(x.reshape(B * T, NN, C), a, m2, w, b.reshape(1, C))
    return y.reshape(B, T, NN, C)


def _make_bnfinal(cnt):
    inv_cnt = 1.0 / float(cnt)

    def body(stats_ref, g_ref, b_ref, scale_ref, shift_ref):
        mean = jnp.sum(stats_ref[0], axis=1, keepdims=True) * inv_cnt
        var = jnp.sum(stats_ref[1], axis=1, keepdims=True) * inv_cnt - mean * mean
        inv = lax.rsqrt(var + 1e-5)
        scale = g_ref[...] * inv
        scale_ref[...] = scale
        shift_ref[...] = b_ref[...] - mean * scale

    return functools.partial(
        _pcall,
        body,
        out_shape=[jax.ShapeDtypeStruct((NN, 1), _F32),
                   jax.ShapeDtypeStruct((NN, 1), _F32)],
    )


def _bnfinal(stats, g, b, cnt):
    return _make_bnfinal(cnt)()(stats, g.reshape(NN, 1), b.reshape(NN, 1))


def _make_head(B, T, C):
    def body(x_ref, scale_ref, shift_ref, w1t_ref, b1_ref, w2t_ref, b2_ref,
             out_ref):
        x = x_ref[0]
        xn = x * scale_ref[...][None] + shift_ref[...][None]
        y1 = _bdot(xn.reshape(T * NN, C), w1t_ref[...]) + b1_ref[...]
        y2 = _bdot(y1, w2t_ref[...]) + b2_ref[...]
        out_ref[0] = y2.reshape(T, NN, 7)

    return functools.partial(
        _pcall,
        body,
        grid=(B,),
        in_specs=[
            pl.BlockSpec((1, T, NN, C), lambda b: (b, 0, 0, 0)),
            pl.BlockSpec((NN, 1), lambda b: (0, 0)),
            pl.BlockSpec((NN, 1), lambda b: (0, 0)),
            pl.BlockSpec((C, 32), lambda b: (0, 0)),
            pl.BlockSpec((1, 32), lambda b: (0, 0)),
            pl.BlockSpec((32, 7), lambda b: (0, 0)),
            pl.BlockSpec((1, 7), lambda b: (0, 0)),
        ],
        out_specs=pl.BlockSpec((1, T, NN, 7), lambda b: (b, 0, 0, 0)),
        out_shape=jax.ShapeDtypeStruct((B, T, NN, 7), _F32),
    )


def kernel(x, edge_index, edge_attr, params):
    row, col = edge_index[0], edge_index[1]
    pad = 16 * _EPT - row.shape[0]
    s_flat, deg = _build_a_sc(jnp.pad(row, (0, pad)),
                              jnp.pad(col, (0, pad)),
                              jnp.pad(edge_attr, (0, pad)))
    a, m2 = _finish_a(s_flat.reshape(_SROWS, _SROW),
                      deg[:NN].reshape(NN, 1),
                      deg[:_SROW].reshape(1, _SROW))

    t = x
    norm = None
    for bname in ("b1", "b2", "b3"):
        p = params[bname]
        t = _tconv(t, p["tc1"], norm=norm, stats=False)
        t = _cheb(t, a, m2, p["cheb_W"], p["cheb_b"])
        t, stats = _tconv(t, p["tc2"], norm=None, stats=True)
        cnt = t.shape[0] * t.shape[1] * t.shape[3]
        norm = _bnfinal(stats, p["bn_g"], p["bn_b"], cnt)

    B, T, _, C = t.shape
    head = _make_head(B, T, C)()
    return head(t, norm[0], norm[1],
                jnp.transpose(params["fc1_W"]),
                params["fc1_b"].reshape(1, 32),
                jnp.transpose(params["fc2_W"]),
                params["fc2_b"].reshape(1, 7))
